# baseline (speedup 1.0000x reference)
import functools

import jax
import jax.numpy as jnp
from jax import lax
from jax.experimental import pallas as pl
from jax.experimental.pallas import tpu as pltpu
from jax.experimental.pallas import tpu_sc as plsc

_L = 16
_NC = 2
_NS = 16
_NW = _NC * _NS


def kernel(x, pos_table):
    B, T, E = x.shape
    N = B * T
    rows_per_w = N // _NW
    R = 16
    nblk = rows_per_w // R
    blk = R * E
    nc = E // _L
    x1 = x.reshape(N * E)

    mesh = plsc.VectorSubcoreMesh(core_axis_name="c", subcore_axis_name="s")

    @functools.partial(
        pl.kernel,
        mesh=mesh,
        out_type=jax.ShapeDtypeStruct((N * E,), jnp.float32),
        scratch_types=[
            pltpu.VMEM((2, blk), jnp.float32),
            pltpu.VMEM((2, blk), jnp.float32),
            pltpu.VMEM((E,), jnp.float32),
            pltpu.SemaphoreType.DMA,
            pltpu.SemaphoreType.DMA,
            pltpu.SemaphoreType.DMA,
            pltpu.SemaphoreType.DMA,
        ],
    )
    def sc_add(x_hbm, pt_hbm, out_hbm, inb, outb, pos_v,
               isem0, isem1, osem0, osem1):
        isems = (isem0, isem1)
        osems = (osem0, osem1)
        wid = lax.axis_index("s") * _NC + lax.axis_index("c")
        base = wid * rows_per_w
        b = base // T
        pltpu.sync_copy(pt_hbm.at[b], pos_v)

        def start_in(g, sl):
            pltpu.async_copy(
                x_hbm.at[pl.ds((base + g * R) * E, blk)], inb.at[sl],
                isems[sl])

        def start_out(g, sl):
            pltpu.async_copy(
                outb.at[sl], out_hbm.at[pl.ds((base + g * R) * E, blk)],
                osems[sl])

        def wait_in(sl):
            pltpu.make_async_copy(
                x_hbm.at[pl.ds(base * E, blk)], inb.at[sl], isems[sl]).wait()

        def wait_out(sl):
            pltpu.make_async_copy(
                outb.at[sl], out_hbm.at[pl.ds(base * E, blk)],
                osems[sl]).wait()

        def add_block(sl):
            for c in range(nc):
                pc = pos_v[pl.ds(c * _L, _L)]

                @plsc.parallel_loop(0, R, unroll=4)
                def rows(r):
                    sli = pl.ds(r * E + c * _L, _L)
                    outb[sl, sli] = inb[sl, sli] + pc

        def do_block(g, sl):
            wait_in(sl)

            @pl.when(g >= 2)
            def _():
                wait_out(sl)

            add_block(sl)
            start_out(g, sl)

            @pl.when(g + 2 < nblk)
            def _():
                start_in(g + 2, sl)

        start_in(0, 0)
        start_in(1, 1)

        def step_body(s, carry):
            for i in range(2):
                do_block(2 * s + i, i)
            return carry

        lax.fori_loop(0, nblk // 2, step_body, 0)
        wait_out(0)
        wait_out(1)

    out = sc_add(x1, pos_table)
    return out.reshape(B, T, E)


# speedup vs baseline: 4.6145x; 4.6145x over previous
import jax
import jax.numpy as jnp
from jax.experimental import pallas as pl


def _add_row_kernel(x_ref, p_ref, o_ref):
    o_ref[...] = x_ref[...] + p_ref[...]


def kernel(x, pos_table):
    B, T, E = x.shape
    TB = 512
    grid = (T // TB,)
    pt3 = pos_table.reshape(pos_table.shape[0], 1, E)
    return pl.pallas_call(
        _add_row_kernel,
        grid=grid,
        in_specs=[
            pl.BlockSpec((B, TB, E), lambda t: (0, t, 0)),
            pl.BlockSpec((B, 1, E), lambda t: (0, 0, 0)),
        ],
        out_specs=pl.BlockSpec((B, TB, E), lambda t: (0, t, 0)),
        out_shape=jax.ShapeDtypeStruct((B, T, E), x.dtype),
    )(x, pt3)


# speedup vs baseline: 6.0602x; 1.3133x over previous
import jax
import jax.numpy as jnp
from jax import lax
from jax.experimental import pallas as pl
from jax.experimental.pallas import tpu as pltpu


def kernel(x, pos_table):
    B, T, E = x.shape
    N = B * T
    RB = 1024
    NBUF = 4
    nblk = N // RB
    blk_per_batch = T // RB
    x2 = x.reshape(N, E)

    def _kern(x_hbm, pt_hbm, o_hbm, pos_v, inb, outb, psem, isems, osems):
        cp = pltpu.make_async_copy(pt_hbm.at[pl.ds(0, B)], pos_v, psem)
        cp.start()
        cp.wait()

        def in_copy(g, sl):
            return pltpu.make_async_copy(
                x_hbm.at[pl.ds(g * RB, RB)], inb.at[sl], isems.at[sl])

        def out_copy(g, sl):
            return pltpu.make_async_copy(
                outb.at[sl], o_hbm.at[pl.ds(g * RB, RB)], osems.at[sl])

        for g in range(NBUF):
            in_copy(g, g).start()

        def loop_body(g, carry):
            sl = lax.rem(g, NBUF)
            in_copy(g, sl).wait()

            @pl.when(g >= NBUF)
            def _():
                out_copy(g - NBUF, sl).wait()

            b = g // blk_per_batch
            outb[sl] = inb[sl] + pos_v[pl.ds(b, 1), :]
            out_copy(g, sl).start()

            @pl.when(g + NBUF < nblk)
            def _():
                in_copy(g + NBUF, sl).start()

            return carry

        lax.fori_loop(0, nblk, loop_body, 0)
        for i in range(NBUF):
            g = nblk - NBUF + i
            out_copy(g, g % NBUF).wait()

    out = pl.pallas_call(
        _kern,
        grid=(),
        in_specs=[
            pl.BlockSpec(memory_space=pl.ANY),
            pl.BlockSpec(memory_space=pl.ANY),
        ],
        out_specs=pl.BlockSpec(memory_space=pl.ANY),
        out_shape=jax.ShapeDtypeStruct((N, E), x.dtype),
        scratch_shapes=[
            pltpu.VMEM((B, E), jnp.float32),
            pltpu.VMEM((NBUF, RB, E), jnp.float32),
            pltpu.VMEM((NBUF, RB, E), jnp.float32),
            pltpu.SemaphoreType.DMA,
            pltpu.SemaphoreType.DMA((NBUF,)),
            pltpu.SemaphoreType.DMA((NBUF,)),
        ],
    )(x2, pos_table)
    return out.reshape(B, T, E)


# speedup vs baseline: 6.0607x; 1.0001x over previous
import jax
import jax.numpy as jnp
from jax import lax
from jax.experimental import pallas as pl
from jax.experimental.pallas import tpu as pltpu


def kernel(x, pos_table):
    B, T, E = x.shape
    N = B * T
    RB = 512
    NBUF = 8
    nblk = N // RB
    blk_per_batch = T // RB
    x2 = x.reshape(N, E)

    def _kern(x_hbm, pt_hbm, o_hbm, pos_v, inb, outb, psem, isems, osems):
        cp = pltpu.make_async_copy(pt_hbm.at[pl.ds(0, B)], pos_v, psem)
        cp.start()
        cp.wait()

        def in_copy(g, sl):
            return pltpu.make_async_copy(
                x_hbm.at[pl.ds(g * RB, RB)], inb.at[sl], isems.at[sl])

        def out_copy(g, sl):
            return pltpu.make_async_copy(
                outb.at[sl], o_hbm.at[pl.ds(g * RB, RB)], osems.at[sl])

        for g in range(NBUF):
            in_copy(g, g).start()

        def loop_body(g, carry):
            sl = lax.rem(g, NBUF)
            in_copy(g, sl).wait()

            @pl.when(g >= NBUF)
            def _():
                out_copy(g - NBUF, sl).wait()

            b = g // blk_per_batch
            outb[sl] = inb[sl] + pos_v[pl.ds(b, 1), :]
            out_copy(g, sl).start()

            @pl.when(g + NBUF < nblk)
            def _():
                in_copy(g + NBUF, sl).start()

            return carry

        lax.fori_loop(0, nblk, loop_body, 0)
        for i in range(NBUF):
            g = nblk - NBUF + i
            out_copy(g, g % NBUF).wait()

    out = pl.pallas_call(
        _kern,
        grid=(),
        in_specs=[
            pl.BlockSpec(memory_space=pl.ANY),
            pl.BlockSpec(memory_space=pl.ANY),
        ],
        out_specs=pl.BlockSpec(memory_space=pl.ANY),
        out_shape=jax.ShapeDtypeStruct((N, E), x.dtype),
        scratch_shapes=[
            pltpu.VMEM((B, E), jnp.float32),
            pltpu.VMEM((NBUF, RB, E), jnp.float32),
            pltpu.VMEM((NBUF, RB, E), jnp.float32),
            pltpu.SemaphoreType.DMA,
            pltpu.SemaphoreType.DMA((NBUF,)),
            pltpu.SemaphoreType.DMA((NBUF,)),
        ],
    )(x2, pos_table)
    return out.reshape(B, T, E)


# speedup vs baseline: 6.0626x; 1.0003x over previous
import jax
import jax.numpy as jnp
from jax import lax
from jax.experimental import pallas as pl
from jax.experimental.pallas import tpu as pltpu


def kernel(x, pos_table):
    B, T, E = x.shape
    N = B * T
    RB = 2048
    NBUF = 3
    nblk = N // RB
    blk_per_batch = T // RB
    x2 = x.reshape(N, E)

    def _kern(x_hbm, pt_hbm, o_hbm, pos_v, inb, outb, psem, isems, osems):
        cp = pltpu.make_async_copy(pt_hbm.at[pl.ds(0, B)], pos_v, psem)
        cp.start()
        cp.wait()

        def in_copy(g, sl):
            return pltpu.make_async_copy(
                x_hbm.at[pl.ds(g * RB, RB)], inb.at[sl], isems.at[sl])

        def out_copy(g, sl):
            return pltpu.make_async_copy(
                outb.at[sl], o_hbm.at[pl.ds(g * RB, RB)], osems.at[sl])

        for g in range(NBUF):
            in_copy(g, g).start()

        def loop_body(g, carry):
            sl = lax.rem(g, NBUF)
            in_copy(g, sl).wait()

            @pl.when(g >= NBUF)
            def _():
                out_copy(g - NBUF, sl).wait()

            b = g // blk_per_batch
            outb[sl] = inb[sl] + pos_v[pl.ds(b, 1), :]
            out_copy(g, sl).start()

            @pl.when(g + NBUF < nblk)
            def _():
                in_copy(g + NBUF, sl).start()

            return carry

        lax.fori_loop(0, nblk, loop_body, 0)
        for i in range(NBUF):
            g = nblk - NBUF + i
            out_copy(g, g % NBUF).wait()

    out = pl.pallas_call(
        _kern,
        grid=(),
        in_specs=[
            pl.BlockSpec(memory_space=pl.ANY),
            pl.BlockSpec(memory_space=pl.ANY),
        ],
        out_specs=pl.BlockSpec(memory_space=pl.ANY),
        out_shape=jax.ShapeDtypeStruct((N, E), x.dtype),
        scratch_shapes=[
            pltpu.VMEM((B, E), jnp.float32),
            pltpu.VMEM((NBUF, RB, E), jnp.float32),
            pltpu.VMEM((NBUF, RB, E), jnp.float32),
            pltpu.SemaphoreType.DMA,
            pltpu.SemaphoreType.DMA((NBUF,)),
            pltpu.SemaphoreType.DMA((NBUF,)),
        ],
    )(x2, pos_table)
    return out.reshape(B, T, E)
